# baseline (speedup 1.0000x reference)
import functools
import math

import jax
import jax.numpy as jnp
from jax import lax
from jax.experimental import pallas as pl
from jax.experimental.pallas import tpu as pltpu
from jax.experimental.pallas import tpu_sc as plsc

D_MODEL = 1024
BATCH = 4
SEQ_LEN = 2048
N_ROWS = BATCH * SEQ_LEN
SCALE = math.sqrt(D_MODEL)
LANES = 16

NUM_CORES = 2
NUM_SUBCORES = 16
NUM_WORKERS = NUM_CORES * NUM_SUBCORES
SPW = SEQ_LEN // NUM_WORKERS
CS = 8
CROWS = CS * BATCH
NCHUNKS = SPW // CS
RPW = SPW * BATCH
UNROLL = 8


def _sc_embed(table, idx1d, pe2d):
    mesh = plsc.VectorSubcoreMesh(core_axis_name="core", subcore_axis_name="subcore")

    @functools.partial(
        pl.kernel,
        out_type=jax.ShapeDtypeStruct((N_ROWS, D_MODEL), jnp.float32),
        mesh=mesh,
        scratch_types=[
            pltpu.VMEM((RPW,), jnp.int32),
            pltpu.VMEM((CROWS, D_MODEL), jnp.float32),
            pltpu.VMEM((CROWS, D_MODEL), jnp.float32),
            pltpu.VMEM((CROWS, D_MODEL), jnp.float32),
            pltpu.VMEM((CS, D_MODEL), jnp.float32),
            pltpu.VMEM((CS, D_MODEL), jnp.float32),
            pltpu.VMEM((CS, D_MODEL), jnp.float32),
            pltpu.SemaphoreType.DMA((3,)),
            pltpu.SemaphoreType.DMA((3,)),
            pltpu.SemaphoreType.DMA((3,)),
        ],
    )
    def kern(table_hbm, idx_hbm, pe_hbm, out_hbm,
             idx_v, rows0, rows1, rows2, pe0, pe1, pe2, gsem, psem, ssem):
        wid = lax.axis_index("core") * NUM_SUBCORES + lax.axis_index("subcore")
        base = wid * RPW
        s_lo = wid * SPW
        pltpu.sync_copy(idx_hbm.at[pl.ds(base, RPW)], idx_v)

        rows = (rows0, rows1, rows2)
        pes = (pe0, pe1, pe2)

        def issue(k):
            b = k % 3
            g = pltpu.async_copy(
                table_hbm.at[idx_v.at[pl.ds(k * CROWS, CROWS)]], rows[b],
                gsem.at[b])
            p = pltpu.async_copy(
                pe_hbm.at[pl.ds(s_lo + k * CS, CS)], pes[b], psem.at[b])
            return g, p

        def compute(rbuf, pbuf):
            @pl.loop(0, CS)
            def _row(t):
                @plsc.parallel_loop(0, D_MODEL, step=LANES, unroll=UNROLL)
                def _col(c):
                    sl = pl.ds(c, LANES)
                    pv = pbuf[t, sl]
                    for bb in range(BATCH):
                        r = bb * CS + t
                        rbuf[r, sl] = rbuf[r, sl] * SCALE + pv

        in_flight = {0: issue(0), 1: issue(1), 2: issue(2)}
        stores = {}
        for k in range(NCHUNKS):
            b = k % 3
            g, p = in_flight.pop(k)
            g.wait()
            p.wait()
            compute(rows[b], pes[b])
            stores[k] = [
                pltpu.async_copy(
                    rows[b].at[pl.ds(bb * CS, CS)],
                    out_hbm.at[pl.ds(bb * SEQ_LEN + s_lo + k * CS, CS)],
                    ssem.at[b])
                for bb in range(BATCH)
            ]
            if k + 2 < NCHUNKS and k >= 1:
                for h in stores[k - 1]:
                    h.wait()
                in_flight[k + 2] = issue(k + 2)
        for k in (NCHUNKS - 2, NCHUNKS - 1):
            for h in stores[k]:
                h.wait()

    return kern(table, idx1d, pe2d)


def kernel(encoded_words, embed_table, pe):
    idx1d = (encoded_words.astype(jnp.int32)
             .reshape(BATCH, NUM_WORKERS, NCHUNKS, CS)
             .transpose(1, 2, 0, 3)
             .reshape(N_ROWS))
    pe2d = pe.reshape(SEQ_LEN, D_MODEL)
    out = _sc_embed(embed_table, idx1d, pe2d)
    return out.reshape(BATCH, SEQ_LEN, D_MODEL)


# speedup vs baseline: 1.0082x; 1.0082x over previous
import functools
import math

import jax
import jax.numpy as jnp
from jax import lax
from jax.experimental import pallas as pl
from jax.experimental.pallas import tpu as pltpu
from jax.experimental.pallas import tpu_sc as plsc

D_MODEL = 1024
BATCH = 4
SEQ_LEN = 2048
N_ROWS = BATCH * SEQ_LEN
SCALE = math.sqrt(D_MODEL)
LANES = 16

NUM_CORES = 2
NUM_SUBCORES = 16
NUM_WORKERS = NUM_CORES * NUM_SUBCORES
SPW = SEQ_LEN // NUM_WORKERS
CS = 8
CROWS = CS * BATCH
NCHUNKS = SPW // CS
RPW = SPW * BATCH
UNROLL = 8


def _sc_embed(table, idx1d, pe2d):
    mesh = plsc.VectorSubcoreMesh(core_axis_name="core", subcore_axis_name="subcore")

    @functools.partial(
        pl.kernel,
        out_type=jax.ShapeDtypeStruct((N_ROWS, D_MODEL), jnp.float32),
        mesh=mesh,
        scratch_types=[
            pltpu.VMEM((RPW,), jnp.int32),
            pltpu.VMEM((CROWS, D_MODEL), jnp.float32),
            pltpu.VMEM((CROWS, D_MODEL), jnp.float32),
            pltpu.VMEM((CROWS, D_MODEL), jnp.float32),
            pltpu.VMEM((CS, D_MODEL), jnp.float32),
            pltpu.VMEM((CS, D_MODEL), jnp.float32),
            pltpu.VMEM((CS, D_MODEL), jnp.float32),
            pltpu.SemaphoreType.DMA((3,)),
            pltpu.SemaphoreType.DMA((3,)),
            pltpu.SemaphoreType.DMA((3,)),
        ],
    )
    def kern(table_hbm, idx_hbm, pe_hbm, out_hbm,
             idx_v, rows0, rows1, rows2, pe0, pe1, pe2, gsem, psem, ssem):
        wid = lax.axis_index("core") * NUM_SUBCORES + lax.axis_index("subcore")
        base = wid * RPW
        s_lo = wid * SPW
        pltpu.sync_copy(idx_hbm.at[pl.ds(base, RPW)], idx_v)

        rows = (rows0, rows1, rows2)
        pes = (pe0, pe1, pe2)

        def issue(k):
            b = k % 3
            g = pltpu.async_copy(
                table_hbm.at[idx_v.at[pl.ds(k * CROWS, CROWS)]], rows[b],
                gsem.at[b])
            p = pltpu.async_copy(
                pe_hbm.at[pl.ds(s_lo + k * CS, CS)], pes[b], psem.at[b])
            return g, p

        def compute(rbuf, pbuf):
            @pl.loop(0, CS)
            def _row(t):
                @plsc.parallel_loop(0, D_MODEL, step=LANES, unroll=UNROLL)
                def _col(c):
                    sl = pl.ds(c, LANES)
                    pv = pbuf[t, sl]
                    for bb in range(BATCH):
                        r = bb * CS + t
                        rbuf[r, sl] = rbuf[r, sl] * SCALE + pv

        in_flight = {0: issue(0), 1: issue(1)}
        stores = {}
        for k in range(NCHUNKS):
            b = k % 3
            g, p = in_flight.pop(k)
            g.wait()
            p.wait()
            compute(rows[b], pes[b])
            stores[k] = [
                pltpu.async_copy(
                    rows[b].at[pl.ds(bb * CS, CS)],
                    out_hbm.at[pl.ds(bb * SEQ_LEN + s_lo + k * CS, CS)],
                    ssem.at[b])
                for bb in range(BATCH)
            ]
            if k + 2 < NCHUNKS:
                if k >= 1:
                    for h in stores[k - 1]:
                        h.wait()
                in_flight[k + 2] = issue(k + 2)
        for k in (NCHUNKS - 2, NCHUNKS - 1):
            for h in stores[k]:
                h.wait()

    return kern(table, idx1d, pe2d)


def kernel(encoded_words, embed_table, pe):
    idx1d = (encoded_words.astype(jnp.int32)
             .reshape(BATCH, NUM_WORKERS, NCHUNKS, CS)
             .transpose(1, 2, 0, 3)
             .reshape(N_ROWS))
    pe2d = pe.reshape(SEQ_LEN, D_MODEL)
    out = _sc_embed(embed_table, idx1d, pe2d)
    return out.reshape(BATCH, SEQ_LEN, D_MODEL)
